# baseline (speedup 1.0000x reference)
import jax
import jax.numpy as jnp
from jax import lax
from jax.experimental import pallas as pl
from jax.experimental.pallas import tpu as pltpu
from jax.experimental.pallas import tpu_sc as plsc

H = 32
W = 32
D = 256
B = 32
F = 2 * D
LANES = 16


def _pos_body(row_hbm, col_hbm, pos_hbm, rowv, build_v):
    c = lax.axis_index("c")
    s = lax.axis_index("s")
    wid = c * 16 + s

    pltpu.sync_copy(col_hbm, build_v.at[:, pl.ds(0, D)])

    pltpu.sync_copy(row_hbm.at[pl.ds(wid, 1)], rowv)

    vs = [rowv[0, pl.ds(j * LANES, LANES)] for j in range(D // LANES)]

    def st(i, carry):
        for j in range(D // LANES):
            build_v[i, pl.ds(D + j * LANES, LANES)] = vs[j]
        return carry

    lax.fori_loop(0, W, st, 0)

    pltpu.sync_copy(build_v, pos_hbm.at[pl.ds(wid * W, W)])


_pos_sc = pl.kernel(
    _pos_body,
    out_type=jax.ShapeDtypeStruct((H * W, F), jnp.float32),
    mesh=plsc.VectorSubcoreMesh(core_axis_name="c", subcore_axis_name="s"),
    scratch_types=[
        pltpu.VMEM((1, D), jnp.float32),
        pltpu.VMEM((W, F), jnp.float32),
    ],
)


NSEM = 8
NSRC = 4


def _bc_body(pos_hbm, out_hbm, pos_v, sem_in, *sems):
    fetches = [
        pltpu.make_async_copy(pos_hbm, pos_v.at[k], sem_in) for k in range(NSRC)
    ]
    for cp in fetches:
        cp.start()
    for cp in fetches:
        cp.wait()
    copies = [
        pltpu.make_async_copy(pos_v.at[b % NSRC], out_hbm.at[b], sems[b % NSEM])
        for b in range(B)
    ]
    for cp in copies:
        cp.start()
    for cp in copies:
        cp.wait()


_bc_tc = pl.pallas_call(
    _bc_body,
    out_shape=jax.ShapeDtypeStruct((B, H * W, F), jnp.float32),
    in_specs=[pl.BlockSpec(memory_space=pl.ANY)],
    out_specs=pl.BlockSpec(memory_space=pl.ANY),
    scratch_shapes=[
        pltpu.VMEM((NSRC, H * W, F), jnp.float32),
        pltpu.SemaphoreType.DMA,
    ]
    + [pltpu.SemaphoreType.DMA] * NSEM,
)


def kernel(x, row_w, col_w):
    del x
    pos = _pos_sc(row_w, col_w)
    return _bc_tc(pos)


# speedup vs baseline: 1.0780x; 1.0780x over previous
import jax
import jax.numpy as jnp
from jax import lax
from jax.experimental import pallas as pl
from jax.experimental.pallas import tpu as pltpu
from jax.experimental.pallas import tpu_sc as plsc

H = 32
W = 32
D = 256
B = 32
F = 2 * D
LANES = 16
NBC = 16


def _pos_body(row_hbm, col_hbm, out_hbm, row2_v, build_v, sem):
    c = lax.axis_index("c")
    s = lax.axis_index("s")

    pltpu.sync_copy(col_hbm, build_v.at[pl.ds(0, W), pl.ds(0, D)])
    pltpu.sync_copy(col_hbm, build_v.at[pl.ds(W, W), pl.ds(0, D)])

    pltpu.sync_copy(row_hbm.at[pl.ds(2 * s, 2)], row2_v)

    for r in range(2):
        vs = [row2_v[r, pl.ds(j * LANES, LANES)] for j in range(D // LANES)]

        def st(i, carry, vs=vs, r=r):
            for j in range(D // LANES):
                build_v[r * W + i, pl.ds(D + j * LANES, LANES)] = vs[j]
            return carry

        lax.fori_loop(0, W, st, 0)

    copies = [
        pltpu.async_copy(
            build_v,
            out_hbm.at[c * NBC + j, pl.ds(s * 64, 64)],
            sem,
        )
        for j in range(NBC)
    ]
    for cp in copies:
        cp.wait()


_pos_kernel = pl.kernel(
    _pos_body,
    out_type=jax.ShapeDtypeStruct((B, H * W, F), jnp.float32),
    mesh=plsc.VectorSubcoreMesh(core_axis_name="c", subcore_axis_name="s"),
    scratch_types=[
        pltpu.VMEM((2, D), jnp.float32),
        pltpu.VMEM((64, F), jnp.float32),
        pltpu.SemaphoreType.DMA,
    ],
)


def kernel(x, row_w, col_w):
    del x
    return _pos_kernel(row_w, col_w)


# speedup vs baseline: 1.1421x; 1.0594x over previous
import jax
import jax.numpy as jnp
from jax import lax
from jax.experimental import pallas as pl
from jax.experimental.pallas import tpu as pltpu
from jax.experimental.pallas import tpu_sc as plsc

H = 32
W = 32
D = 256
B = 32
F = 2 * D
LANES = 16


def _pos_body(row_hbm, col_hbm, out_hbm, rowv, build_v, sem):
    c = lax.axis_index("c")
    s = lax.axis_index("s")
    wid = c * 16 + s

    pltpu.sync_copy(col_hbm, build_v.at[:, pl.ds(0, D)])

    pltpu.sync_copy(row_hbm.at[pl.ds(wid, 1)], rowv)

    vs = [rowv[0, pl.ds(j * LANES, LANES)] for j in range(D // LANES)]

    def st(i, carry):
        for j in range(D // LANES):
            build_v[i, pl.ds(D + j * LANES, LANES)] = vs[j]
        return carry

    lax.fori_loop(0, W, st, 0)

    copies = [
        pltpu.async_copy(
            build_v,
            out_hbm.at[lax.rem(wid + j, B), pl.ds(wid * W, W)],
            sem,
        )
        for j in range(B)
    ]
    for cp in copies:
        cp.wait()


_pos_kernel = pl.kernel(
    _pos_body,
    out_type=jax.ShapeDtypeStruct((B, H * W, F), jnp.float32),
    mesh=plsc.VectorSubcoreMesh(core_axis_name="c", subcore_axis_name="s"),
    scratch_types=[
        pltpu.VMEM((1, D), jnp.float32),
        pltpu.VMEM((W, F), jnp.float32),
        pltpu.SemaphoreType.DMA,
    ],
)


def kernel(x, row_w, col_w):
    del x
    return _pos_kernel(row_w, col_w)
